# Initial kernel scaffold; baseline (speedup 1.0000x reference)
import jax
import jax.numpy as jnp
from jax.experimental import pallas as pl


def kernel(x, edge_index, W1, b1, W2, b2):
    raise NotImplementedError("write your pallas kernel here")


# speedup vs baseline: 17.5620x; 17.5620x over previous
import functools

import jax
import jax.numpy as jnp
from jax import lax
from jax.experimental import pallas as pl
from jax.experimental.pallas import tpu as pltpu
from jax.experimental.pallas import tpu_sc as plsc

_N = 10000
_E = 320000
_D = 128
_H = 128
_C = 16

_NC = 2
_NS = 16
_NW = _NC * _NS
_CH = 80
_EPT = _E // _NW
_NCHUNK = _EPT // _CH
_RPS = 624
_RLAST = _N - (_NS - 1) * _RPS

_mesh = plsc.VectorSubcoreMesh(core_axis_name="c", subcore_axis_name="s")


def _sliced_copy(s, src, src_base, dst, dst_base):
    @pl.when(s < _NS - 1)
    def _():
        o = pl.multiple_of(s * _RPS, 16)
        pltpu.sync_copy(src.at[pl.ds(src_base + o, _RPS)],
                        dst.at[pl.ds(dst_base + o, _RPS)])

    @pl.when(s == _NS - 1)
    def _():
        o = (_NS - 1) * _RPS
        pltpu.sync_copy(src.at[pl.ds(src_base + o, _RLAST)],
                        dst.at[pl.ds(dst_base + o, _RLAST)])


def _deg_body(dst_hbm, zeros_hbm, ones_hbm, out_hbm, idx_all, ones_v, acc):
    c = lax.axis_index("c")
    s = lax.axis_index("s")
    wid = c * _NS + s
    pltpu.sync_copy(ones_hbm, ones_v)
    _sliced_copy(s, zeros_hbm, 0, acc, 0)
    pltpu.sync_copy(dst_hbm.at[wid], idx_all)
    plsc.subcore_barrier()

    def body(i, carry):
        pltpu.sync_copy(ones_v, acc.at[idx_all.at[i]], add=True)
        return carry

    lax.fori_loop(0, _NCHUNK, body, 0)
    plsc.subcore_barrier()
    _sliced_copy(s, acc, 0, out_hbm, c * _N)


_deg_call = functools.partial(
    pl.kernel,
    out_type=jax.ShapeDtypeStruct((2 * _N, _D), jnp.float32),
    mesh=_mesh,
    scratch_types=[
        pltpu.VMEM((_NCHUNK, _CH), jnp.int32),
        pltpu.VMEM((_CH, _D), jnp.float32),
        pltpu.VMEM_SHARED((_N, _D), jnp.float32),
    ],
)(_deg_body)


def _agg_body(feat_hbm, src_hbm, dst_hbm, out_hbm, src_all, dst_all,
              rows_v, acc, sem):
    c = lax.axis_index("c")
    s = lax.axis_index("s")
    wid = c * _NS + s
    _sliced_copy(s, feat_hbm, 0, acc, 0)
    pltpu.sync_copy(src_hbm.at[wid], src_all)
    pltpu.sync_copy(dst_hbm.at[wid], dst_all)
    plsc.subcore_barrier()

    def body(i, carry):
        pltpu.async_copy(feat_hbm.at[src_all.at[i]], rows_v, sem).wait()
        pltpu.sync_copy(rows_v, acc.at[dst_all.at[i]], add=True)
        return carry

    lax.fori_loop(0, _NCHUNK, body, 0)
    plsc.subcore_barrier()
    _sliced_copy(s, acc, 0, out_hbm, c * _N)


_agg_call = functools.partial(
    pl.kernel,
    out_type=jax.ShapeDtypeStruct((2 * _N, _D), jnp.float32),
    mesh=_mesh,
    scratch_types=[
        pltpu.VMEM((_NCHUNK, _CH), jnp.int32),
        pltpu.VMEM((_NCHUNK, _CH), jnp.int32),
        pltpu.VMEM((_CH, _D), jnp.float32),
        pltpu.VMEM_SHARED((_N, _D), jnp.float32),
        pltpu.SemaphoreType.DMA,
    ],
)(_agg_body)


def _tc1_body(x_ref, w1_ref, degp_ref, hs_ref, dinv_ref):
    deg = degp_ref[: _N, 0:1] + degp_ref[_N:, 0:1] + 1.0
    dinv = lax.rsqrt(deg)
    h1 = jnp.dot(x_ref[...], w1_ref[...], preferred_element_type=jnp.float32)
    hs_ref[...] = h1 * dinv
    dinv_ref[...] = dinv


_tc1 = pl.pallas_call(
    _tc1_body,
    out_shape=(jax.ShapeDtypeStruct((_N, _D), jnp.float32),
               jax.ShapeDtypeStruct((_N, 1), jnp.float32)),
)


def _tc2_body(p_ref, hs_ref, dinv_ref, b1_ref, f2_ref):
    agg = p_ref[: _N] + p_ref[_N:] - hs_ref[...]
    f2_ref[...] = jnp.maximum(agg * dinv_ref[...] + b1_ref[...],
                              0.0) * dinv_ref[...]


_tc2 = pl.pallas_call(
    _tc2_body,
    out_shape=jax.ShapeDtypeStruct((_N, _H), jnp.float32),
)


def _tc3_body(q_ref, f2_ref, dinv_ref, w2_ref, b2_ref, out_ref):
    agg = (q_ref[: _N] + q_ref[_N:] - f2_ref[...]) * dinv_ref[...]
    out_ref[...] = jnp.dot(agg, w2_ref[...],
                           preferred_element_type=jnp.float32) + b2_ref[...]


_tc3 = pl.pallas_call(
    _tc3_body,
    out_shape=jax.ShapeDtypeStruct((_N, _C), jnp.float32),
)


def kernel(x, edge_index, W1, b1, W2, b2):
    src = edge_index[0].astype(jnp.int32).reshape(_NW, _NCHUNK, _CH)
    dst = edge_index[1].astype(jnp.int32).reshape(_NW, _NCHUNK, _CH)
    zeros128 = jnp.zeros((_N, _D), jnp.float32)
    ones128 = jnp.ones((_CH, _D), jnp.float32)

    degp = _deg_call(dst, zeros128, ones128)
    hs, dinv = _tc1(x, W1, degp)
    p = _agg_call(hs, src, dst)
    f2 = _tc2(p, hs, dinv, b1.reshape(1, _H))
    q = _agg_call(f2, src, dst)
    return _tc3(q, f2, dinv, W2, b2.reshape(1, _C))


# speedup vs baseline: 21.3181x; 1.2139x over previous
import functools

import jax
import jax.numpy as jnp
from jax import lax
from jax.experimental import pallas as pl
from jax.experimental.pallas import tpu as pltpu
from jax.experimental.pallas import tpu_sc as plsc

_N = 10000
_E = 320000
_D = 128
_H = 128
_C = 16

_NC = 2
_NS = 16
_NW = _NC * _NS
_CH = 80
_EPT = _E // _NW
_NCHUNK = _EPT // _CH
_RPS = 624
_RLAST = _N - (_NS - 1) * _RPS

_mesh = plsc.VectorSubcoreMesh(core_axis_name="c", subcore_axis_name="s")


def _sliced_copy(s, src, src_base, dst, dst_base):
    @pl.when(s < _NS - 1)
    def _():
        o = pl.multiple_of(s * _RPS, 16)
        pltpu.sync_copy(src.at[pl.ds(src_base + o, _RPS)],
                        dst.at[pl.ds(dst_base + o, _RPS)])

    @pl.when(s == _NS - 1)
    def _():
        o = (_NS - 1) * _RPS
        pltpu.sync_copy(src.at[pl.ds(src_base + o, _RLAST)],
                        dst.at[pl.ds(dst_base + o, _RLAST)])


_DEGK = 8


def _deg_body(dst_hbm, zeros_hbm, ones_hbm, out_hbm, idx_all, ones_v, acc,
              sems):
    c = lax.axis_index("c")
    s = lax.axis_index("s")
    wid = c * _NS + s
    pltpu.sync_copy(ones_hbm, ones_v)
    _sliced_copy(s, zeros_hbm, 0, acc, 0)
    pltpu.sync_copy(dst_hbm.at[wid], idx_all)
    plsc.subcore_barrier()

    for i in range(_DEGK):
        pltpu.async_copy(ones_v, acc.at[idx_all.at[i]], sems, add=True)

    def body(i, carry):
        pltpu.make_async_copy(ones_v, acc.at[idx_all.at[0]], sems).wait()
        pltpu.async_copy(ones_v, acc.at[idx_all.at[i + _DEGK]], sems,
                         add=True)
        return carry

    lax.fori_loop(0, _NCHUNK - _DEGK, body, 0)
    for i in range(_DEGK):
        pltpu.make_async_copy(ones_v, acc.at[idx_all.at[0]], sems).wait()
    plsc.subcore_barrier()
    _sliced_copy(s, acc, 0, out_hbm, c * _N)


_deg_call = functools.partial(
    pl.kernel,
    out_type=jax.ShapeDtypeStruct((2 * _N, _D), jnp.float32),
    mesh=_mesh,
    scratch_types=[
        pltpu.VMEM((_NCHUNK, _CH), jnp.int32),
        pltpu.VMEM((_CH, _D), jnp.float32),
        pltpu.VMEM_SHARED((_N, _D), jnp.float32),
        pltpu.SemaphoreType.DMA,
    ],
)(_deg_body)


def _agg_body(feat_hbm, packed_hbm, out_hbm, packed_all, s0, s1, d0, d1,
              r0, r1, acc, semg, sems):
    c = lax.axis_index("c")
    s = lax.axis_index("s")
    wid = c * _NS + s
    _sliced_copy(s, feat_hbm, 0, acc, 0)
    pltpu.sync_copy(packed_hbm.at[pl.ds(wid * _EPT, _EPT)], packed_all)
    plsc.subcore_barrier()

    def unpack(i, sbuf, dbuf):
        for k in range(_CH // 16):
            w = packed_all[pl.ds(i * _CH + k * 16, 16)]
            sbuf[pl.ds(k * 16, 16)] = w >> 14
            dbuf[pl.ds(k * 16, 16)] = w & 0x3FFF

    def g(sbuf, buf):
        pltpu.async_copy(feat_hbm.at[sbuf], buf, semg)

    def sc(dbuf, buf):
        pltpu.async_copy(buf, acc.at[dbuf], sems, add=True)

    def wait_g():
        pltpu.make_async_copy(feat_hbm.at[s0], r0, semg).wait()

    def wait_s():
        pltpu.make_async_copy(r0, acc.at[d0], sems).wait()

    unpack(0, s0, d0)
    g(s0, r0)

    def body(j, carry):
        a = 2 * j
        wait_g()
        sc(d0, r0)

        @pl.when(j > 0)
        def _():
            wait_s()

        unpack(a + 1, s1, d1)
        g(s1, r1)
        wait_g()
        sc(d1, r1)
        wait_s()
        unpack(a + 2, s0, d0)
        g(s0, r0)
        return carry

    lax.fori_loop(0, (_NCHUNK - 1) // 2, body, 0)
    wait_g()
    sc(d0, r0)
    wait_s()
    wait_s()
    plsc.subcore_barrier()
    _sliced_copy(s, acc, 0, out_hbm, c * _N)


_agg_call = functools.partial(
    pl.kernel,
    out_type=jax.ShapeDtypeStruct((2 * _N, _D), jnp.float32),
    mesh=_mesh,
    scratch_types=[
        pltpu.VMEM((_EPT,), jnp.int32),
        pltpu.VMEM((_CH,), jnp.int32),
        pltpu.VMEM((_CH,), jnp.int32),
        pltpu.VMEM((_CH,), jnp.int32),
        pltpu.VMEM((_CH,), jnp.int32),
        pltpu.VMEM((_CH, _D), jnp.float32),
        pltpu.VMEM((_CH, _D), jnp.float32),
        pltpu.VMEM_SHARED((_N, _D), jnp.float32),
        pltpu.SemaphoreType.DMA,
        pltpu.SemaphoreType.DMA,
    ],
)(_agg_body)


def _tc1_body(x_ref, w1_ref, degp_ref, hs_ref, dinv_ref):
    deg = degp_ref[: _N, 0:1] + degp_ref[_N:, 0:1] + 1.0
    dinv = lax.rsqrt(deg)
    h1 = jnp.dot(x_ref[...], w1_ref[...], preferred_element_type=jnp.float32)
    hs_ref[...] = h1 * dinv
    dinv_ref[...] = dinv


_tc1 = pl.pallas_call(
    _tc1_body,
    out_shape=(jax.ShapeDtypeStruct((_N, _D), jnp.float32),
               jax.ShapeDtypeStruct((_N, 1), jnp.float32)),
)


def _tc2_body(p_ref, hs_ref, dinv_ref, b1_ref, f2_ref):
    agg = p_ref[: _N] + p_ref[_N:] - hs_ref[...]
    f2_ref[...] = jnp.maximum(agg * dinv_ref[...] + b1_ref[...],
                              0.0) * dinv_ref[...]


_tc2 = pl.pallas_call(
    _tc2_body,
    out_shape=jax.ShapeDtypeStruct((_N, _H), jnp.float32),
)


def _tc3_body(q_ref, f2_ref, dinv_ref, w2_ref, b2_ref, out_ref):
    agg = (q_ref[: _N] + q_ref[_N:] - f2_ref[...]) * dinv_ref[...]
    out_ref[...] = jnp.dot(agg, w2_ref[...],
                           preferred_element_type=jnp.float32) + b2_ref[...]


_tc3 = pl.pallas_call(
    _tc3_body,
    out_shape=jax.ShapeDtypeStruct((_N, _C), jnp.float32),
)


def kernel(x, edge_index, W1, b1, W2, b2):
    src_f = edge_index[0].astype(jnp.int32)
    dst_f = edge_index[1].astype(jnp.int32)
    dst = dst_f.reshape(_NW, _NCHUNK, _CH)
    packed = (src_f << 14) | dst_f
    zeros128 = jnp.zeros((_N, _D), jnp.float32)
    ones128 = jnp.ones((_CH, _D), jnp.float32)

    degp = _deg_call(dst, zeros128, ones128)
    hs, dinv = _tc1(x, W1, degp)
    p = _agg_call(hs, packed)
    f2 = _tc2(p, hs, dinv, b1.reshape(1, _H))
    q = _agg_call(f2, packed)
    return _tc3(q, f2, dinv, W2, b2.reshape(1, _C))


# speedup vs baseline: 25.6608x; 1.2037x over previous
import functools

import jax
import jax.numpy as jnp
from jax import lax
from jax.experimental import pallas as pl
from jax.experimental.pallas import tpu as pltpu
from jax.experimental.pallas import tpu_sc as plsc

_N = 10000
_E = 320000
_D = 128
_H = 128
_C = 16

_NC = 2
_NS = 16
_NW = _NC * _NS
_CH = 80
_EPT = _E // _NW
_NCHUNK = _EPT // _CH
_RPS = 624
_RLAST = _N - (_NS - 1) * _RPS

_mesh = plsc.VectorSubcoreMesh(core_axis_name="c", subcore_axis_name="s")


def _sliced_copy(s, src, src_base, dst, dst_base):
    @pl.when(s < _NS - 1)
    def _():
        o = pl.multiple_of(s * _RPS, 16)
        pltpu.sync_copy(src.at[pl.ds(src_base + o, _RPS)],
                        dst.at[pl.ds(dst_base + o, _RPS)])

    @pl.when(s == _NS - 1)
    def _():
        o = (_NS - 1) * _RPS
        pltpu.sync_copy(src.at[pl.ds(src_base + o, _RLAST)],
                        dst.at[pl.ds(dst_base + o, _RLAST)])


_DEGK = 8


def _deg_body(dst_hbm, zeros_hbm, ones_hbm, out_hbm, idx_all, ones_v, acc,
              sems):
    c = lax.axis_index("c")
    s = lax.axis_index("s")
    wid = c * _NS + s
    pltpu.sync_copy(ones_hbm, ones_v)
    _sliced_copy(s, zeros_hbm, 0, acc, 0)
    pltpu.sync_copy(dst_hbm.at[wid], idx_all)
    plsc.subcore_barrier()

    for i in range(_DEGK):
        pltpu.async_copy(ones_v, acc.at[idx_all.at[i]], sems, add=True)

    def body(i, carry):
        pltpu.make_async_copy(ones_v, acc.at[idx_all.at[0]], sems).wait()
        pltpu.async_copy(ones_v, acc.at[idx_all.at[i + _DEGK]], sems,
                         add=True)
        return carry

    lax.fori_loop(0, _NCHUNK - _DEGK, body, 0)
    for i in range(_DEGK):
        pltpu.make_async_copy(ones_v, acc.at[idx_all.at[0]], sems).wait()
    plsc.subcore_barrier()
    _sliced_copy(s, acc, 0, out_hbm, c * _N)


_deg_call = functools.partial(
    pl.kernel,
    out_type=jax.ShapeDtypeStruct((2 * _N, _D), jnp.float32),
    mesh=_mesh,
    scratch_types=[
        pltpu.VMEM((_NCHUNK, _CH), jnp.int32),
        pltpu.VMEM((_CH, _D), jnp.float32),
        pltpu.VMEM_SHARED((_N, _D), jnp.float32),
        pltpu.SemaphoreType.DMA,
    ],
)(_deg_body)


def _agg_body(feat_hbm, src_hbm, dst_hbm, out_hbm, s0, s1, s2, d0, d1, d2,
              r0, r1, r2, acc, semg, sems, semi):
    c = lax.axis_index("c")
    s = lax.axis_index("s")
    wid = c * _NS + s
    sb = [s0, s1, s2]
    db = [d0, d1, d2]
    rb = [r0, r1, r2]
    base = wid * _EPT
    _sliced_copy(s, feat_hbm, 0, acc, 0)
    for m in range(3):
        pltpu.sync_copy(src_hbm.at[pl.ds(base + m * _CH, _CH)], sb[m])
        pltpu.sync_copy(dst_hbm.at[pl.ds(base + m * _CH, _CH)], db[m])
    plsc.subcore_barrier()

    def li(m, sbuf, dbuf):
        o = pl.multiple_of(base + m * _CH, 8)
        pltpu.async_copy(src_hbm.at[pl.ds(o, _CH)], sbuf, semi)
        pltpu.async_copy(dst_hbm.at[pl.ds(o, _CH)], dbuf, semi)

    def g(b):
        pltpu.async_copy(feat_hbm.at[sb[b]], rb[b], semg)

    def sc(b):
        pltpu.async_copy(rb[b], acc.at[db[b]], sems, add=True)

    def wait_g():
        pltpu.make_async_copy(feat_hbm.at[s0], r0, semg).wait()

    def wait_s():
        pltpu.make_async_copy(r0, acc.at[d0], sems).wait()

    def wait_i():
        pltpu.make_async_copy(src_hbm.at[pl.ds(0, _CH)], s0, semi).wait()

    for b in range(3):
        g(b)

    def body(j, carry):
        a = 3 * j
        for b in range(3):
            i = a + b
            wait_g()
            sc(b)
            wait_s()

            @pl.when(i + 3 < _NCHUNK)
            def _():
                li(i + 3, sb[b], db[b])

            @pl.when(jnp.logical_and(i >= 1, i + 2 < _NCHUNK))
            def _():
                wait_i()
                wait_i()
                g((b + 2) % 3)
        return carry

    lax.fori_loop(0, _NCHUNK // 3, body, 0)
    for b in range(_NCHUNK - 3 * (_NCHUNK // 3)):
        wait_g()
        sc(b)
        wait_s()
    plsc.subcore_barrier()
    _sliced_copy(s, acc, 0, out_hbm, c * _N)


_agg_call = functools.partial(
    pl.kernel,
    out_type=jax.ShapeDtypeStruct((2 * _N, _D), jnp.float32),
    mesh=_mesh,
    scratch_types=[
        pltpu.VMEM((_CH,), jnp.int32),
        pltpu.VMEM((_CH,), jnp.int32),
        pltpu.VMEM((_CH,), jnp.int32),
        pltpu.VMEM((_CH,), jnp.int32),
        pltpu.VMEM((_CH,), jnp.int32),
        pltpu.VMEM((_CH,), jnp.int32),
        pltpu.VMEM((_CH, _D), jnp.float32),
        pltpu.VMEM((_CH, _D), jnp.float32),
        pltpu.VMEM((_CH, _D), jnp.float32),
        pltpu.VMEM_SHARED((_N, _D), jnp.float32),
        pltpu.SemaphoreType.DMA,
        pltpu.SemaphoreType.DMA,
        pltpu.SemaphoreType.DMA,
    ],
)(_agg_body)


def _tc1_body(x_ref, w1_ref, degp_ref, hs_ref, dinv_ref):
    deg = degp_ref[: _N, 0:1] + degp_ref[_N:, 0:1] + 1.0
    dinv = lax.rsqrt(deg)
    h1 = jnp.dot(x_ref[...], w1_ref[...], preferred_element_type=jnp.float32)
    hs_ref[...] = h1 * dinv
    dinv_ref[...] = dinv


_tc1 = pl.pallas_call(
    _tc1_body,
    out_shape=(jax.ShapeDtypeStruct((_N, _D), jnp.float32),
               jax.ShapeDtypeStruct((_N, 1), jnp.float32)),
)


def _tc2_body(p_ref, hs_ref, dinv_ref, b1_ref, f2_ref):
    agg = p_ref[: _N] + p_ref[_N:] - hs_ref[...]
    f2_ref[...] = jnp.maximum(agg * dinv_ref[...] + b1_ref[...],
                              0.0) * dinv_ref[...]


_tc2 = pl.pallas_call(
    _tc2_body,
    out_shape=jax.ShapeDtypeStruct((_N, _H), jnp.float32),
)


def _tc3_body(q_ref, f2_ref, dinv_ref, w2_ref, b2_ref, out_ref):
    agg = (q_ref[: _N] + q_ref[_N:] - f2_ref[...]) * dinv_ref[...]
    out_ref[...] = jnp.dot(agg, w2_ref[...],
                           preferred_element_type=jnp.float32) + b2_ref[...]


_tc3 = pl.pallas_call(
    _tc3_body,
    out_shape=jax.ShapeDtypeStruct((_N, _C), jnp.float32),
)


def kernel(x, edge_index, W1, b1, W2, b2):
    src_f = edge_index[0].astype(jnp.int32)
    dst_f = edge_index[1].astype(jnp.int32)
    dst = dst_f.reshape(_NW, _NCHUNK, _CH)
    zeros128 = jnp.zeros((_N, _D), jnp.float32)
    ones128 = jnp.ones((_CH, _D), jnp.float32)

    degp = _deg_call(dst, zeros128, ones128)
    hs, dinv = _tc1(x, W1, degp)
    p = _agg_call(hs, src_f, dst_f)
    f2 = _tc2(p, hs, dinv, b1.reshape(1, _H))
    q = _agg_call(f2, src_f, dst_f)
    return _tc3(q, f2, dinv, W2, b2.reshape(1, _C))


# speedup vs baseline: 27.1171x; 1.0568x over previous
import functools

import jax
import jax.numpy as jnp
from jax import lax
from jax.experimental import pallas as pl
from jax.experimental.pallas import tpu as pltpu
from jax.experimental.pallas import tpu_sc as plsc

_N = 10000
_E = 320000
_D = 128
_H = 128
_C = 16

_NC = 2
_NS = 16
_NW = _NC * _NS
_CH = 80
_EPT = _E // _NW
_NCHUNK = _EPT // _CH
_RPS = 624
_RLAST = _N - (_NS - 1) * _RPS

_mesh = plsc.VectorSubcoreMesh(core_axis_name="c", subcore_axis_name="s")


def _sliced_copy(s, src, src_base, dst, dst_base):
    @pl.when(s < _NS - 1)
    def _():
        o = pl.multiple_of(s * _RPS, 16)
        pltpu.sync_copy(src.at[pl.ds(src_base + o, _RPS)],
                        dst.at[pl.ds(dst_base + o, _RPS)])

    @pl.when(s == _NS - 1)
    def _():
        o = (_NS - 1) * _RPS
        pltpu.sync_copy(src.at[pl.ds(src_base + o, _RLAST)],
                        dst.at[pl.ds(dst_base + o, _RLAST)])


_DEGK = 8


def _deg_body(dst_hbm, zeros_hbm, ones_hbm, out_hbm, idx_all, ones_v, acc,
              sems):
    c = lax.axis_index("c")
    s = lax.axis_index("s")
    wid = c * _NS + s
    pltpu.sync_copy(ones_hbm, ones_v)
    _sliced_copy(s, zeros_hbm, 0, acc, 0)
    pltpu.sync_copy(dst_hbm.at[wid], idx_all)
    plsc.subcore_barrier()

    for i in range(_DEGK):
        pltpu.async_copy(ones_v, acc.at[idx_all.at[i]], sems, add=True)

    def body(i, carry):
        pltpu.make_async_copy(ones_v, acc.at[idx_all.at[0]], sems).wait()
        pltpu.async_copy(ones_v, acc.at[idx_all.at[i + _DEGK]], sems,
                         add=True)
        return carry

    lax.fori_loop(0, _NCHUNK - _DEGK, body, 0)
    for i in range(_DEGK):
        pltpu.make_async_copy(ones_v, acc.at[idx_all.at[0]], sems).wait()
    plsc.subcore_barrier()
    _sliced_copy(s, acc, 0, out_hbm, c * _N)


_deg_call = functools.partial(
    pl.kernel,
    out_type=jax.ShapeDtypeStruct((2 * _N, _D), jnp.float32),
    mesh=_mesh,
    scratch_types=[
        pltpu.VMEM((_NCHUNK, _CH), jnp.int32),
        pltpu.VMEM((_CH, _D), jnp.float32),
        pltpu.VMEM_SHARED((_N, _D), jnp.float32),
        pltpu.SemaphoreType.DMA,
    ],
)(_deg_body)


def _agg_body(feat_hbm, src_hbm, dst_hbm, out_hbm, s0, s1, s2, s3, s4, s5,
              d0, d1, d2, d3, d4, d5, r0, r1, r2, acc, semg, sems, semi):
    c = lax.axis_index("c")
    s = lax.axis_index("s")
    wid = c * _NS + s
    sb = [s0, s1, s2, s3, s4, s5]
    db = [d0, d1, d2, d3, d4, d5]
    rb = [r0, r1, r2]
    base = wid * _EPT
    _sliced_copy(s, feat_hbm, 0, acc, 0)
    for m in range(3):
        pltpu.sync_copy(src_hbm.at[pl.ds(base + m * _CH, _CH)], sb[m])
        pltpu.sync_copy(dst_hbm.at[pl.ds(base + m * _CH, _CH)], db[m])
    plsc.subcore_barrier()

    def li(m, b):
        o = pl.multiple_of(base + m * _CH, 8)
        pltpu.async_copy(src_hbm.at[pl.ds(o, _CH)], sb[b], semi)
        pltpu.async_copy(dst_hbm.at[pl.ds(o, _CH)], db[b], semi)

    def g(b6, b3):
        pltpu.async_copy(feat_hbm.at[sb[b6]], rb[b3], semg)

    def sc(b6, b3):
        pltpu.async_copy(rb[b3], acc.at[db[b6]], sems, add=True)

    def wait_g():
        pltpu.make_async_copy(feat_hbm.at[s0], r0, semg).wait()

    def wait_s():
        pltpu.make_async_copy(r0, acc.at[d0], sems).wait()

    def wait_i():
        pltpu.make_async_copy(src_hbm.at[pl.ds(0, _CH)], s0, semi).wait()

    for b in range(3):
        g(b, b)

    def step(i, b):
        wait_g()
        sc(b, b % 3)

        @pl.when(i >= 1)
        def _():
            wait_s()

        @pl.when(i + 3 < _NCHUNK)
        def _():
            li(i + 3, (b + 3) % 6)

        @pl.when(jnp.logical_and(i >= 1, i + 2 < _NCHUNK))
        def _():
            wait_i()
            wait_i()
            g((b + 2) % 6, (b + 2) % 3)

    def body(j, carry):
        a = 6 * j
        for b in range(6):
            step(a + b, b)
        return carry

    lax.fori_loop(0, _NCHUNK // 6, body, 0)
    for b in range(_NCHUNK - 6 * (_NCHUNK // 6)):
        step(jnp.int32(6 * (_NCHUNK // 6) + b), b)
    wait_s()
    plsc.subcore_barrier()
    _sliced_copy(s, acc, 0, out_hbm, c * _N)


_agg_call = functools.partial(
    pl.kernel,
    out_type=jax.ShapeDtypeStruct((2 * _N, _D), jnp.float32),
    mesh=_mesh,
    scratch_types=(
        [pltpu.VMEM((_CH,), jnp.int32)] * 12
        + [pltpu.VMEM((_CH, _D), jnp.float32)] * 3
        + [pltpu.VMEM_SHARED((_N, _D), jnp.float32)]
        + [pltpu.SemaphoreType.DMA] * 3
    ),
)(_agg_body)


def _tc1_body(x_ref, w1_ref, degp_ref, hs_ref, dinv_ref):
    deg = degp_ref[: _N, 0:1] + degp_ref[_N:, 0:1] + 1.0
    dinv = lax.rsqrt(deg)
    h1 = jnp.dot(x_ref[...], w1_ref[...], preferred_element_type=jnp.float32)
    hs_ref[...] = h1 * dinv
    dinv_ref[...] = dinv


_tc1 = pl.pallas_call(
    _tc1_body,
    out_shape=(jax.ShapeDtypeStruct((_N, _D), jnp.float32),
               jax.ShapeDtypeStruct((_N, 1), jnp.float32)),
)


def _tc2_body(p_ref, hs_ref, dinv_ref, b1_ref, f2_ref):
    agg = p_ref[: _N] + p_ref[_N:] - hs_ref[...]
    f2_ref[...] = jnp.maximum(agg * dinv_ref[...] + b1_ref[...],
                              0.0) * dinv_ref[...]


_tc2 = pl.pallas_call(
    _tc2_body,
    out_shape=jax.ShapeDtypeStruct((_N, _H), jnp.float32),
)


def _tc3_body(q_ref, f2_ref, dinv_ref, w2_ref, b2_ref, out_ref):
    agg = (q_ref[: _N] + q_ref[_N:] - f2_ref[...]) * dinv_ref[...]
    out_ref[...] = jnp.dot(agg, w2_ref[...],
                           preferred_element_type=jnp.float32) + b2_ref[...]


_tc3 = pl.pallas_call(
    _tc3_body,
    out_shape=jax.ShapeDtypeStruct((_N, _C), jnp.float32),
)


def kernel(x, edge_index, W1, b1, W2, b2):
    src_f = edge_index[0].astype(jnp.int32)
    dst_f = edge_index[1].astype(jnp.int32)
    dst = dst_f.reshape(_NW, _NCHUNK, _CH)
    zeros128 = jnp.zeros((_N, _D), jnp.float32)
    ones128 = jnp.ones((_CH, _D), jnp.float32)

    degp = _deg_call(dst, zeros128, ones128)
    hs, dinv = _tc1(x, W1, degp)
    p = _agg_call(hs, src_f, dst_f)
    f2 = _tc2(p, hs, dinv, b1.reshape(1, _H))
    q = _agg_call(f2, src_f, dst_f)
    return _tc3(q, f2, dinv, W2, b2.reshape(1, _C))
